# Initial kernel scaffold; baseline (speedup 1.0000x reference)
import jax
import jax.numpy as jnp
from jax.experimental import pallas as pl


def kernel(x, params, edge_index):
    raise NotImplementedError("write your pallas kernel here")


# speedup vs baseline: 5.1513x; 5.1513x over previous
import functools

import jax
import jax.numpy as jnp
from jax import lax
from jax.experimental import pallas as pl
from jax.experimental.pallas import tpu as pltpu
from jax.experimental.pallas import tpu_sc as plsc

N = 50000
E = 800000
FEAT = 128
H = 64
HH = 32

NP = 50176
ROWS_PER_TILE = NP // 16
BLK = 512
GRID = NP // BLK

E_PAD = 16 * 50176
AGG_ROWS = E_PAD // 128
AGG_ROWS_PER_TILE = AGG_ROWS // 16
AGG_K = 4
AGG_NB = AGG_ROWS_PER_TILE // AGG_K

E_CNT = 32 * 25600
CNT_ROWS = E_CNT // 128
CNT_ROWS_PER_WORKER = CNT_ROWS // 32
CNT_K = 8
CNT_NB = CNT_ROWS_PER_WORKER // CNT_K

def _mesh():
  return plsc.VectorSubcoreMesh(core_axis_name="c", subcore_axis_name="s")

DOT = functools.partial(lax.dot_general, precision=lax.Precision.HIGHEST)


def _mm(a, b):
  return DOT(a, b, (((1,), (0,)), ((), ())), preferred_element_type=jnp.float32)


def _sc_cnt_body(dst_hbm, zeros_hbm, out_hbm, dbuf, ones_v, acc, sem):
  c = lax.axis_index("c")
  s = lax.axis_index("s")
  w = c * 16 + s

  def fill_ones(i, _):
    ones_v[i] = jnp.ones((16,), jnp.float32)
    return _

  lax.fori_loop(0, 128, fill_ones, None)

  pltpu.sync_copy(zeros_hbm.at[pl.ds(s * ROWS_PER_TILE, ROWS_PER_TILE)],
                  acc.at[pl.ds(s * ROWS_PER_TILE, ROWS_PER_TILE)])
  plsc.subcore_barrier()

  def body(b, carry):
    base = w * CNT_ROWS_PER_WORKER + b * CNT_K
    pltpu.sync_copy(dst_hbm.at[pl.ds(base, CNT_K)], dbuf)
    for j in range(CNT_K):
      pltpu.sync_copy(ones_v, acc.at[dbuf.at[j]], add=True)
    return carry

  lax.fori_loop(0, CNT_NB, body, None)
  plsc.subcore_barrier()
  pltpu.sync_copy(acc.at[pl.ds(s * ROWS_PER_TILE, ROWS_PER_TILE)],
                  out_hbm.at[c, pl.ds(s * ROWS_PER_TILE, ROWS_PER_TILE)])


def _sc_cnt(dst_cnt, zeros16):
  k = pl.kernel(
      _sc_cnt_body,
      out_type=jax.ShapeDtypeStruct((2, NP, 16), jnp.float32),
      mesh=_mesh(),
      compiler_params=pltpu.CompilerParams(use_tc_tiling_on_sc=False),
      scratch_types=[
          pltpu.VMEM((CNT_K, 128), jnp.int32),
          pltpu.VMEM((128, 16), jnp.float32),
          pltpu.VMEM_SHARED((NP, 16), jnp.float32),
          pltpu.SemaphoreType.DMA,
      ],
  )
  return k(dst_cnt, zeros16)


def _sc_agg_body(h0_hbm, h1_hbm, src_hbm, dst_hbm, zeros_hbm, out_hbm,
                 sbuf, dbuf, rows, acc, sem):
  c = lax.axis_index("c")
  s = lax.axis_index("s")

  pltpu.sync_copy(zeros_hbm.at[pl.ds(s * ROWS_PER_TILE, ROWS_PER_TILE)],
                  acc.at[pl.ds(s * ROWS_PER_TILE, ROWS_PER_TILE)])
  plsc.subcore_barrier()

  def run(tab):
    def body(b, carry):
      base = s * AGG_ROWS_PER_TILE + b * AGG_K
      pltpu.sync_copy(src_hbm.at[pl.ds(base, AGG_K)], sbuf)
      pltpu.sync_copy(dst_hbm.at[pl.ds(base, AGG_K)], dbuf)
      descs = [
          pltpu.async_copy(tab.at[sbuf.at[j]], rows.at[j], sem)
          for j in range(AGG_K)
      ]
      for d in descs:
        d.wait()
      for j in range(AGG_K):
        pltpu.sync_copy(rows.at[j], acc.at[dbuf.at[j]], add=True)
      return carry

    lax.fori_loop(0, AGG_NB, body, None)

  @pl.when(c == 0)
  def _():
    run(h0_hbm)

  @pl.when(c == 1)
  def _():
    run(h1_hbm)

  plsc.subcore_barrier()
  pltpu.sync_copy(acc.at[pl.ds(s * ROWS_PER_TILE, ROWS_PER_TILE)],
                  out_hbm.at[c, pl.ds(s * ROWS_PER_TILE, ROWS_PER_TILE)])


def _sc_agg(h0, h1, src_p, dst_p, zeros32):
  k = pl.kernel(
      _sc_agg_body,
      out_type=jax.ShapeDtypeStruct((2, NP, HH), jnp.float32),
      mesh=_mesh(),
      compiler_params=pltpu.CompilerParams(use_tc_tiling_on_sc=False),
      scratch_types=[
          pltpu.VMEM((AGG_K, 128), jnp.int32),
          pltpu.VMEM((AGG_K, 128), jnp.int32),
          pltpu.VMEM((AGG_K, 128, HH), jnp.float32),
          pltpu.VMEM_SHARED((NP, HH), jnp.float32),
          pltpu.SemaphoreType.DMA,
      ],
  )
  return k(h0, h1, src_p, dst_p, zeros32)


def _proj_body(x_ref, w_ref, b_ref, h0_ref, h1_ref):
  y = _mm(x_ref[...], w_ref[...]) + b_ref[...]
  h0_ref[...] = y[:, :HH]
  h1_ref[...] = y[:, HH:]


def _tc_proj(x_p, w, b):
  return pl.pallas_call(
      _proj_body,
      grid=(GRID,),
      in_specs=[
          pl.BlockSpec((BLK, FEAT), lambda i: (i, 0)),
          pl.BlockSpec((FEAT, H), lambda i: (0, 0)),
          pl.BlockSpec((1, H), lambda i: (0, 0)),
      ],
      out_specs=[
          pl.BlockSpec((BLK, HH), lambda i: (i, 0)),
          pl.BlockSpec((BLK, HH), lambda i: (i, 0)),
      ],
      out_shape=[
          jax.ShapeDtypeStruct((NP, HH), jnp.float32),
          jax.ShapeDtypeStruct((NP, HH), jnp.float32),
      ],
  )(x_p, w, b)


def _layer_body(h0_ref, h1_ref, a0_ref, a1_ref, c0_ref, c1_ref,
                wl_ref, wr_ref, bb_ref, g_ref, bn_ref, *out_refs, last):
  h = jnp.concatenate([h0_ref[...], h1_ref[...]], axis=1)
  cnt = c0_ref[0] + c1_ref[0]
  cm = jnp.maximum(cnt[:, :1], 1.0)
  agg = jnp.concatenate([a0_ref[0], a1_ref[0]], axis=1) / cm
  t = _mm(agg, wl_ref[...]) + _mm(h, wr_ref[...]) + bb_ref[...]
  m = jnp.mean(t, axis=1, keepdims=True)
  d = t - m
  v = jnp.mean(d * d, axis=1, keepdims=True)
  t = d * lax.rsqrt(v + 1e-5) * g_ref[...] + bn_ref[...]
  t = jnp.maximum(t, 0.0)
  hn = h + t
  out_refs[0][...] = hn[:, :HH]
  out_refs[1][...] = hn[:, HH:]
  if last:
    out_refs[2][...] = hn
    i = pl.program_id(0)
    row = i * BLK + lax.broadcasted_iota(jnp.int32, (BLK, 1), 0)
    hm = jnp.where(row < N, hn, 0.0)
    part = jnp.sum(hm, axis=0, keepdims=True)

    @pl.when(i == 0)
    def _():
      out_refs[3][...] = jnp.zeros((1, H), jnp.float32)

    out_refs[3][...] += part


def _tc_layer(h0, h1, agg2, cnt2, wl, wr, bb, g, bn, last):
  out_specs = [
      pl.BlockSpec((BLK, HH), lambda i: (i, 0)),
      pl.BlockSpec((BLK, HH), lambda i: (i, 0)),
  ]
  out_shape = [
      jax.ShapeDtypeStruct((NP, HH), jnp.float32),
      jax.ShapeDtypeStruct((NP, HH), jnp.float32),
  ]
  if last:
    out_specs += [
        pl.BlockSpec((BLK, H), lambda i: (i, 0)),
        pl.BlockSpec((1, H), lambda i: (0, 0)),
    ]
    out_shape += [
        jax.ShapeDtypeStruct((NP, H), jnp.float32),
        jax.ShapeDtypeStruct((1, H), jnp.float32),
    ]
  return pl.pallas_call(
      functools.partial(_layer_body, last=last),
      grid=(GRID,),
      in_specs=[
          pl.BlockSpec((BLK, HH), lambda i: (i, 0)),
          pl.BlockSpec((BLK, HH), lambda i: (i, 0)),
          pl.BlockSpec((1, BLK, HH), lambda i: (0, i, 0)),
          pl.BlockSpec((1, BLK, HH), lambda i: (1, i, 0)),
          pl.BlockSpec((1, BLK, 16), lambda i: (0, i, 0)),
          pl.BlockSpec((1, BLK, 16), lambda i: (1, i, 0)),
          pl.BlockSpec((H, H), lambda i: (0, 0)),
          pl.BlockSpec((H, H), lambda i: (0, 0)),
          pl.BlockSpec((1, H), lambda i: (0, 0)),
          pl.BlockSpec((1, H), lambda i: (0, 0)),
          pl.BlockSpec((1, H), lambda i: (0, 0)),
      ],
      out_specs=out_specs,
      out_shape=out_shape,
  )(h0, h1, agg2, agg2, cnt2, cnt2, wl, wr, bb, g, bn)


def _gmlp_body(gs_ref, w1_ref, b1_ref, w2_ref, b2_ref, iw1_ref, ib1_ref,
               iw2_ref, ib2_ref, g_ref, imp_ref):
  g0 = jnp.broadcast_to(gs_ref[...] * (1.0 / N), (8, H))
  a = jnp.maximum(_mm(g0, w1_ref[...]) + b1_ref[...], 0.0)
  g = _mm(a, w2_ref[...]) + b2_ref[...]
  ib = jnp.maximum(_mm(g, iw1_ref[...]) + ib1_ref[...], 0.0)
  imp = _mm(ib, iw2_ref[...]) + ib2_ref[...]
  g_ref[...] = g[:1]
  imp_ref[...] = imp[:1]


def _tc_gmlp(gsum, p):
  return pl.pallas_call(
      _gmlp_body,
      out_shape=[
          jax.ShapeDtypeStruct((1, H), jnp.float32),
          jax.ShapeDtypeStruct((1, 3), jnp.float32),
      ],
  )(gsum, p['gp_W1'], p['gp_b1'].reshape(1, -1), p['gp_W2'],
    p['gp_b2'].reshape(1, -1), p['im_W1'], p['im_b1'].reshape(1, -1),
    p['im_W2'], p['im_b2'].reshape(1, -1))


def _heads_body(h_ref, g_ref, sw1_ref, sb1_ref, sw2_ref, sb2_ref,
                ewh_ref, ewg_ref, eb1_ref, ew2_ref, eb2_ref,
                sev_ref, sp_ref):
  h = h_ref[...]
  s1 = jnp.maximum(_mm(h, sw1_ref[...]) + sb1_ref[...], 0.0)
  sev_ref[...] = jnp.maximum(_mm(s1, sw2_ref[...]) + sb2_ref[...], 0.0)
  gg = _mm(g_ref[...], ewg_ref[...]) + eb1_ref[...]
  e1 = jnp.maximum(_mm(h, ewh_ref[...]) + gg, 0.0)
  z = _mm(e1, ew2_ref[...]) + eb2_ref[...]
  sp_ref[...] = 1.0 / (1.0 + jnp.exp(-z))


def _tc_heads(hf, g, p):
  return pl.pallas_call(
      _heads_body,
      grid=(GRID,),
      in_specs=[
          pl.BlockSpec((BLK, H), lambda i: (i, 0)),
          pl.BlockSpec((1, H), lambda i: (0, 0)),
          pl.BlockSpec((H, HH), lambda i: (0, 0)),
          pl.BlockSpec((1, HH), lambda i: (0, 0)),
          pl.BlockSpec((HH, 1), lambda i: (0, 0)),
          pl.BlockSpec((1, 1), lambda i: (0, 0)),
          pl.BlockSpec((H, HH), lambda i: (0, 0)),
          pl.BlockSpec((H, HH), lambda i: (0, 0)),
          pl.BlockSpec((1, HH), lambda i: (0, 0)),
          pl.BlockSpec((HH, 1), lambda i: (0, 0)),
          pl.BlockSpec((1, 1), lambda i: (0, 0)),
      ],
      out_specs=[
          pl.BlockSpec((BLK, 1), lambda i: (i, 0)),
          pl.BlockSpec((BLK, 1), lambda i: (i, 0)),
      ],
      out_shape=[
          jax.ShapeDtypeStruct((NP, 1), jnp.float32),
          jax.ShapeDtypeStruct((NP, 1), jnp.float32),
      ],
  )(hf, g, p['sv_W1'], p['sv_b1'].reshape(1, -1), p['sv_W2'],
    p['sv_b2'].reshape(1, -1), p['se_W1'][:H], p['se_W1'][H:],
    p['se_b1'].reshape(1, -1), p['se_W2'], p['se_b2'].reshape(1, -1))


def kernel(x, params, edge_index):
  p = params
  src = edge_index[0]
  dst = edge_index[1]

  pad_a = E_PAD - E
  src_p = jnp.concatenate(
      [src, (jnp.arange(pad_a, dtype=jnp.int32) * 127) % N]).reshape(
          AGG_ROWS, 128)
  dst_p = jnp.concatenate(
      [dst, N + jnp.arange(pad_a, dtype=jnp.int32) % (NP - N)]).reshape(
          AGG_ROWS, 128)
  pad_c = E_CNT - E
  dst_c = jnp.concatenate(
      [dst, N + jnp.arange(pad_c, dtype=jnp.int32) % (NP - N)]).reshape(
          CNT_ROWS, 128)

  x_p = jnp.pad(x, ((0, NP - N), (0, 0)))
  zeros32 = jnp.zeros((NP, HH), jnp.float32)
  zeros16 = jnp.zeros((NP, 16), jnp.float32)

  cnt2 = _sc_cnt(dst_c, zeros16)
  h0, h1 = _tc_proj(x_p, p['in_W'], p['in_b'].reshape(1, -1))

  hf = gsum = None
  for i in range(3):
    agg2 = _sc_agg(h0, h1, src_p, dst_p, zeros32)
    bb = (p['l%d_bl' % i] + p['l%d_br' % i]).reshape(1, -1)
    outs = _tc_layer(h0, h1, agg2, cnt2, p['l%d_Wl' % i], p['l%d_Wr' % i],
                     bb, p['l%d_g' % i].reshape(1, -1),
                     p['l%d_bn' % i].reshape(1, -1), last=(i == 2))
    if i == 2:
      h0, h1, hf, gsum = outs
    else:
      h0, h1 = outs

  g, imp = _tc_gmlp(gsum, p)
  sev, sp = _tc_heads(hf, g, p)
  return (sev[:N], imp, sp[:N], hf[:N], g)


# speedup vs baseline: 6.2946x; 1.2219x over previous
import functools

import jax
import jax.numpy as jnp
from jax import lax
from jax.experimental import pallas as pl
from jax.experimental.pallas import tpu as pltpu
from jax.experimental.pallas import tpu_sc as plsc

N = 50000
E = 800000
FEAT = 128
H = 64
HH = 32

NP = 50176
BLK = 512
GRID = NP // BLK

ACC_ROWS = 50048
SLICE = ACC_ROWS // 16
PADR = ACC_ROWS - N

E_PAD = 16 * 50176
EDGES_PER_TILE = E_PAD // 16
AGG_C = 448
AGG_CHUNKS = EDGES_PER_TILE // AGG_C

E_CNT = 32 * 25600
CNT_EPW = E_CNT // 32
CNT_C = 3200
CNT_NB = CNT_EPW // CNT_C

def _mesh():
  return plsc.VectorSubcoreMesh(core_axis_name="c", subcore_axis_name="s")

DOT = functools.partial(lax.dot_general, precision=lax.Precision.HIGHEST)


def _mm(a, b):
  return DOT(a, b, (((1,), (0,)), ((), ())), preferred_element_type=jnp.float32)


def _sc_cnt_body(dst_hbm, zeros_hbm, ones_hbm, out_hbm, dbuf, ones_v, acc, sem):
  c = lax.axis_index("c")
  s = lax.axis_index("s")
  w = c * 16 + s

  pltpu.sync_copy(ones_hbm, ones_v)
  pltpu.sync_copy(zeros_hbm.at[pl.ds(s * SLICE, SLICE)],
                  acc.at[pl.ds(s * SLICE, SLICE)])
  plsc.subcore_barrier()

  def body(b, carry):
    base = w * CNT_EPW + b * CNT_C
    pltpu.sync_copy(dst_hbm.at[pl.ds(base, CNT_C)], dbuf)
    pltpu.sync_copy(ones_v, acc.at[dbuf], add=True)
    return carry

  lax.fori_loop(0, CNT_NB, body, None)
  plsc.subcore_barrier()
  pltpu.sync_copy(acc.at[pl.ds(s * SLICE, SLICE)],
                  out_hbm.at[c, pl.ds(s * SLICE, SLICE)])


def _sc_cnt(dst_cnt, zeros16, ones16):
  k = pl.kernel(
      _sc_cnt_body,
      out_type=jax.ShapeDtypeStruct((2, NP, 16), jnp.float32),
      mesh=_mesh(),
      compiler_params=pltpu.CompilerParams(use_tc_tiling_on_sc=False),
      scratch_types=[
          pltpu.VMEM((CNT_C,), jnp.int32),
          pltpu.VMEM((CNT_C, 16), jnp.float32),
          pltpu.VMEM_SHARED((ACC_ROWS, 16), jnp.float32),
          pltpu.SemaphoreType.DMA,
      ],
  )
  return k(dst_cnt, zeros16, ones16)


def _sc_agg_body(h0_hbm, h1_hbm, src_hbm, dst_hbm, zeros_hbm, out_hbm,
                 sbuf0, dbuf0, rows0, sbuf1, dbuf1, rows1, acc,
                 sem_g0, sem_g1, sem_s0, sem_s1):
  c = lax.axis_index("c")
  s = lax.axis_index("s")

  pltpu.sync_copy(zeros_hbm.at[pl.ds(s * SLICE, SLICE)],
                  acc.at[pl.ds(s * SLICE, SLICE)])
  plsc.subcore_barrier()

  bufs = ((sbuf0, dbuf0, rows0, sem_g0, sem_s0),
          (sbuf1, dbuf1, rows1, sem_g1, sem_s1))

  def run(tab):
    ebase = s * EDGES_PER_TILE
    pltpu.sync_copy(src_hbm.at[pl.ds(ebase, AGG_C)], sbuf0)
    pltpu.sync_copy(dst_hbm.at[pl.ds(ebase, AGG_C)], dbuf0)
    pltpu.async_copy(tab.at[sbuf0], rows0, sem_g0)

    def outer(g2, carry):
      for p in (0, 1):
        sb, db, rw, sg, ss = bufs[p]
        qb, qd, qr, qg, qs = bufs[1 - p]
        i = g2 * 2 + p

        @pl.when(i + 1 < AGG_CHUNKS)
        def _():
          @pl.when(i >= 1)
          def _():
            pltpu.make_async_copy(qr, acc.at[qd], qs).wait()
          nb = ebase + (i + 1) * AGG_C
          pltpu.sync_copy(src_hbm.at[pl.ds(nb, AGG_C)], qb)
          pltpu.sync_copy(dst_hbm.at[pl.ds(nb, AGG_C)], qd)
          pltpu.async_copy(tab.at[qb], qr, qg)

        pltpu.make_async_copy(tab.at[sb], rw, sg).wait()
        pltpu.async_copy(rw, acc.at[db], ss, add=True)
      return carry

    lax.fori_loop(0, AGG_CHUNKS // 2, outer, None)
    pltpu.make_async_copy(rows0, acc.at[dbuf0], sem_s0).wait()
    pltpu.make_async_copy(rows1, acc.at[dbuf1], sem_s1).wait()

  @pl.when(c == 0)
  def _():
    run(h0_hbm)

  @pl.when(c == 1)
  def _():
    run(h1_hbm)

  plsc.subcore_barrier()
  pltpu.sync_copy(acc.at[pl.ds(s * SLICE, SLICE)],
                  out_hbm.at[c, pl.ds(s * SLICE, SLICE)])


def _sc_agg(h0, h1, src_p, dst_p, zeros32):
  k = pl.kernel(
      _sc_agg_body,
      out_type=jax.ShapeDtypeStruct((2, NP, HH), jnp.float32),
      mesh=_mesh(),
      compiler_params=pltpu.CompilerParams(use_tc_tiling_on_sc=False),
      scratch_types=[
          pltpu.VMEM((AGG_C,), jnp.int32),
          pltpu.VMEM((AGG_C,), jnp.int32),
          pltpu.VMEM((AGG_C, HH), jnp.float32),
          pltpu.VMEM((AGG_C,), jnp.int32),
          pltpu.VMEM((AGG_C,), jnp.int32),
          pltpu.VMEM((AGG_C, HH), jnp.float32),
          pltpu.VMEM_SHARED((ACC_ROWS, HH), jnp.float32),
          pltpu.SemaphoreType.DMA,
          pltpu.SemaphoreType.DMA,
          pltpu.SemaphoreType.DMA,
          pltpu.SemaphoreType.DMA,
      ],
  )
  return k(h0, h1, src_p, dst_p, zeros32)


def _proj_body(x_ref, w_ref, b_ref, h0_ref, h1_ref):
  y = _mm(x_ref[...], w_ref[...]) + b_ref[...]
  h0_ref[...] = y[:, :HH]
  h1_ref[...] = y[:, HH:]


def _tc_proj(x_p, w, b):
  return pl.pallas_call(
      _proj_body,
      grid=(GRID,),
      in_specs=[
          pl.BlockSpec((BLK, FEAT), lambda i: (i, 0)),
          pl.BlockSpec((FEAT, H), lambda i: (0, 0)),
          pl.BlockSpec((1, H), lambda i: (0, 0)),
      ],
      out_specs=[
          pl.BlockSpec((BLK, HH), lambda i: (i, 0)),
          pl.BlockSpec((BLK, HH), lambda i: (i, 0)),
      ],
      out_shape=[
          jax.ShapeDtypeStruct((NP, HH), jnp.float32),
          jax.ShapeDtypeStruct((NP, HH), jnp.float32),
      ],
  )(x_p, w, b)


def _layer_body(h0_ref, h1_ref, a0_ref, a1_ref, c0_ref, c1_ref,
                wl_ref, wr_ref, bb_ref, g_ref, bn_ref, *out_refs, last):
  h = jnp.concatenate([h0_ref[...], h1_ref[...]], axis=1)
  cnt = c0_ref[0] + c1_ref[0]
  cm = jnp.maximum(cnt[:, :1], 1.0)
  agg = jnp.concatenate([a0_ref[0], a1_ref[0]], axis=1) / cm
  t = _mm(agg, wl_ref[...]) + _mm(h, wr_ref[...]) + bb_ref[...]
  m = jnp.mean(t, axis=1, keepdims=True)
  d = t - m
  v = jnp.mean(d * d, axis=1, keepdims=True)
  t = d * lax.rsqrt(v + 1e-5) * g_ref[...] + bn_ref[...]
  t = jnp.maximum(t, 0.0)
  hn = h + t
  out_refs[0][...] = hn[:, :HH]
  out_refs[1][...] = hn[:, HH:]
  if last:
    out_refs[2][...] = hn
    i = pl.program_id(0)
    row = i * BLK + lax.broadcasted_iota(jnp.int32, (BLK, 1), 0)
    hm = jnp.where(row < N, hn, 0.0)
    part = jnp.sum(hm, axis=0, keepdims=True)

    @pl.when(i == 0)
    def _():
      out_refs[3][...] = jnp.zeros((1, H), jnp.float32)

    out_refs[3][...] += part


def _tc_layer(h0, h1, agg2, cnt2, wl, wr, bb, g, bn, last):
  out_specs = [
      pl.BlockSpec((BLK, HH), lambda i: (i, 0)),
      pl.BlockSpec((BLK, HH), lambda i: (i, 0)),
  ]
  out_shape = [
      jax.ShapeDtypeStruct((NP, HH), jnp.float32),
      jax.ShapeDtypeStruct((NP, HH), jnp.float32),
  ]
  if last:
    out_specs += [
        pl.BlockSpec((BLK, H), lambda i: (i, 0)),
        pl.BlockSpec((1, H), lambda i: (0, 0)),
    ]
    out_shape += [
        jax.ShapeDtypeStruct((NP, H), jnp.float32),
        jax.ShapeDtypeStruct((1, H), jnp.float32),
    ]
  return pl.pallas_call(
      functools.partial(_layer_body, last=last),
      grid=(GRID,),
      in_specs=[
          pl.BlockSpec((BLK, HH), lambda i: (i, 0)),
          pl.BlockSpec((BLK, HH), lambda i: (i, 0)),
          pl.BlockSpec((1, BLK, HH), lambda i: (0, i, 0)),
          pl.BlockSpec((1, BLK, HH), lambda i: (1, i, 0)),
          pl.BlockSpec((1, BLK, 16), lambda i: (0, i, 0)),
          pl.BlockSpec((1, BLK, 16), lambda i: (1, i, 0)),
          pl.BlockSpec((H, H), lambda i: (0, 0)),
          pl.BlockSpec((H, H), lambda i: (0, 0)),
          pl.BlockSpec((1, H), lambda i: (0, 0)),
          pl.BlockSpec((1, H), lambda i: (0, 0)),
          pl.BlockSpec((1, H), lambda i: (0, 0)),
      ],
      out_specs=out_specs,
      out_shape=out_shape,
  )(h0, h1, agg2, agg2, cnt2, cnt2, wl, wr, bb, g, bn)


def _gmlp_body(gs_ref, w1_ref, b1_ref, w2_ref, b2_ref, iw1_ref, ib1_ref,
               iw2_ref, ib2_ref, g_ref, imp_ref):
  g0 = jnp.broadcast_to(gs_ref[...] * (1.0 / N), (8, H))
  a = jnp.maximum(_mm(g0, w1_ref[...]) + b1_ref[...], 0.0)
  g = _mm(a, w2_ref[...]) + b2_ref[...]
  ib = jnp.maximum(_mm(g, iw1_ref[...]) + ib1_ref[...], 0.0)
  imp = _mm(ib, iw2_ref[...]) + ib2_ref[...]
  g_ref[...] = g[:1]
  imp_ref[...] = imp[:1]


def _tc_gmlp(gsum, p):
  return pl.pallas_call(
      _gmlp_body,
      out_shape=[
          jax.ShapeDtypeStruct((1, H), jnp.float32),
          jax.ShapeDtypeStruct((1, 3), jnp.float32),
      ],
  )(gsum, p['gp_W1'], p['gp_b1'].reshape(1, -1), p['gp_W2'],
    p['gp_b2'].reshape(1, -1), p['im_W1'], p['im_b1'].reshape(1, -1),
    p['im_W2'], p['im_b2'].reshape(1, -1))


def _heads_body(h_ref, g_ref, sw1_ref, sb1_ref, sw2_ref, sb2_ref,
                ewh_ref, ewg_ref, eb1_ref, ew2_ref, eb2_ref,
                sev_ref, sp_ref):
  h = h_ref[...]
  s1 = jnp.maximum(_mm(h, sw1_ref[...]) + sb1_ref[...], 0.0)
  sev_ref[...] = jnp.maximum(_mm(s1, sw2_ref[...]) + sb2_ref[...], 0.0)
  gg = _mm(g_ref[...], ewg_ref[...]) + eb1_ref[...]
  e1 = jnp.maximum(_mm(h, ewh_ref[...]) + gg, 0.0)
  z = _mm(e1, ew2_ref[...]) + eb2_ref[...]
  sp_ref[...] = 1.0 / (1.0 + jnp.exp(-z))


def _tc_heads(hf, g, p):
  return pl.pallas_call(
      _heads_body,
      grid=(GRID,),
      in_specs=[
          pl.BlockSpec((BLK, H), lambda i: (i, 0)),
          pl.BlockSpec((1, H), lambda i: (0, 0)),
          pl.BlockSpec((H, HH), lambda i: (0, 0)),
          pl.BlockSpec((1, HH), lambda i: (0, 0)),
          pl.BlockSpec((HH, 1), lambda i: (0, 0)),
          pl.BlockSpec((1, 1), lambda i: (0, 0)),
          pl.BlockSpec((H, HH), lambda i: (0, 0)),
          pl.BlockSpec((H, HH), lambda i: (0, 0)),
          pl.BlockSpec((1, HH), lambda i: (0, 0)),
          pl.BlockSpec((HH, 1), lambda i: (0, 0)),
          pl.BlockSpec((1, 1), lambda i: (0, 0)),
      ],
      out_specs=[
          pl.BlockSpec((BLK, 1), lambda i: (i, 0)),
          pl.BlockSpec((BLK, 1), lambda i: (i, 0)),
      ],
      out_shape=[
          jax.ShapeDtypeStruct((NP, 1), jnp.float32),
          jax.ShapeDtypeStruct((NP, 1), jnp.float32),
      ],
  )(hf, g, p['sv_W1'], p['sv_b1'].reshape(1, -1), p['sv_W2'],
    p['sv_b2'].reshape(1, -1), p['se_W1'][:H], p['se_W1'][H:],
    p['se_b1'].reshape(1, -1), p['se_W2'], p['se_b2'].reshape(1, -1))


def kernel(x, params, edge_index):
  p = params
  src = edge_index[0]
  dst = edge_index[1]

  pad_a = E_PAD - E
  src_p = jnp.concatenate(
      [src, (jnp.arange(pad_a, dtype=jnp.int32) * 127) % N])
  dst_p = jnp.concatenate(
      [dst, N + jnp.arange(pad_a, dtype=jnp.int32) % PADR])
  pad_c = E_CNT - E
  dst_c = jnp.concatenate(
      [dst, N + jnp.arange(pad_c, dtype=jnp.int32) % PADR])

  x_p = jnp.pad(x, ((0, NP - N), (0, 0)))
  zeros32 = jnp.zeros((ACC_ROWS, HH), jnp.float32)
  zeros16 = jnp.zeros((ACC_ROWS, 16), jnp.float32)
  ones16 = jnp.ones((CNT_C, 16), jnp.float32)

  cnt2 = _sc_cnt(dst_c, zeros16, ones16)
  h0, h1 = _tc_proj(x_p, p['in_W'], p['in_b'].reshape(1, -1))

  hf = gsum = None
  for i in range(3):
    agg2 = _sc_agg(h0, h1, src_p, dst_p, zeros32)
    bb = (p['l%d_bl' % i] + p['l%d_br' % i]).reshape(1, -1)
    outs = _tc_layer(h0, h1, agg2, cnt2, p['l%d_Wl' % i], p['l%d_Wr' % i],
                     bb, p['l%d_g' % i].reshape(1, -1),
                     p['l%d_bn' % i].reshape(1, -1), last=(i == 2))
    if i == 2:
      h0, h1, hf, gsum = outs
    else:
      h0, h1 = outs

  g, imp = _tc_gmlp(gsum, p)
  sev, sp = _tc_heads(hf, g, p)
  return (sev[:N], imp, sp[:N], hf[:N], g)


# speedup vs baseline: 7.8193x; 1.2422x over previous
import functools

import jax
import jax.numpy as jnp
from jax import lax
from jax.experimental import pallas as pl
from jax.experimental.pallas import tpu as pltpu
from jax.experimental.pallas import tpu_sc as plsc

N = 50000
E = 800000
FEAT = 128
H = 64
HH = 32

NP = 50176
BLK = 512
GRID = NP // BLK

ACC_ROWS = 50048
SLICE = ACC_ROWS // 16
PADR = ACC_ROWS - N

AGG_C = 264
AGG_CHUNKS = 192
AGG_UNROLL = 12
EDGES_PER_TILE = AGG_C * AGG_CHUNKS
E_PAD = 16 * EDGES_PER_TILE

E_CNT = 32 * 25600
CNT_EPW = E_CNT // 32
CNT_C = 3200
CNT_NB = CNT_EPW // CNT_C

def _mesh():
  return plsc.VectorSubcoreMesh(core_axis_name="c", subcore_axis_name="s")

DOT = functools.partial(lax.dot_general, precision=lax.Precision.DEFAULT)


def _mm(a, b):
  return DOT(a, b, (((1,), (0,)), ((), ())), preferred_element_type=jnp.float32)


def _sc_cnt_body(dst_hbm, zeros_hbm, ones_hbm, out_hbm, dbuf, ones_v, acc, sem):
  c = lax.axis_index("c")
  s = lax.axis_index("s")
  w = c * 16 + s

  pltpu.sync_copy(ones_hbm, ones_v)
  pltpu.sync_copy(zeros_hbm.at[pl.ds(s * SLICE, SLICE)],
                  acc.at[pl.ds(s * SLICE, SLICE)])
  plsc.subcore_barrier()

  def body(b, carry):
    base = w * CNT_EPW + b * CNT_C
    pltpu.sync_copy(dst_hbm.at[pl.ds(base, CNT_C)], dbuf)
    pltpu.sync_copy(ones_v, acc.at[dbuf], add=True)
    return carry

  lax.fori_loop(0, CNT_NB, body, None)
  plsc.subcore_barrier()
  pltpu.sync_copy(acc.at[pl.ds(s * SLICE, SLICE)],
                  out_hbm.at[c, pl.ds(s * SLICE, SLICE)])


def _sc_cnt(dst_cnt, zeros16, ones16):
  k = pl.kernel(
      _sc_cnt_body,
      out_type=jax.ShapeDtypeStruct((2, NP, 16), jnp.float32),
      mesh=_mesh(),
      compiler_params=pltpu.CompilerParams(use_tc_tiling_on_sc=False),
      scratch_types=[
          pltpu.VMEM((CNT_C,), jnp.int32),
          pltpu.VMEM((CNT_C, 16), jnp.float32),
          pltpu.VMEM_SHARED((ACC_ROWS, 16), jnp.float32),
          pltpu.SemaphoreType.DMA,
      ],
  )
  return k(dst_cnt, zeros16, ones16)


def _sc_agg_body(h0_hbm, h1_hbm, src_hbm, dst_hbm, zeros_hbm, out_hbm,
                 *scr):
  c = lax.axis_index("c")
  s = lax.axis_index("s")
  sbufs, rows, sems_i, sems_g = scr[0:3], scr[3:6], scr[6:9], scr[9:12]
  dbufs, sems_s = scr[12:16], scr[16:20]
  acc = scr[20]

  pltpu.sync_copy(zeros_hbm.at[pl.ds(s * SLICE, SLICE)],
                  acc.at[pl.ds(s * SLICE, SLICE)])
  plsc.subcore_barrier()

  def run(tab):
    ebase = s * EDGES_PER_TILE

    def issue_idx(i, u):
      nb = ebase + i * AGG_C
      pltpu.async_copy(src_hbm.at[pl.ds(nb, AGG_C)], sbufs[u % 3],
                       sems_i[u % 3])
      pltpu.async_copy(dst_hbm.at[pl.ds(nb, AGG_C)], dbufs[u % 4],
                       sems_i[u % 3])

    def wait_idx(u):
      pltpu.make_async_copy(src_hbm.at[pl.ds(0, AGG_C)], sbufs[u % 3],
                            sems_i[u % 3]).wait()
      pltpu.make_async_copy(src_hbm.at[pl.ds(0, AGG_C)], dbufs[u % 4],
                            sems_i[u % 3]).wait()

    def issue_gather(u):
      pltpu.async_copy(tab.at[sbufs[u % 3]], rows[u % 3], sems_g[u % 3])

    def wait_gather(u):
      pltpu.make_async_copy(tab.at[sbufs[u % 3]], rows[u % 3],
                            sems_g[u % 3]).wait()

    def issue_scatter(u):
      pltpu.async_copy(rows[u % 3], acc.at[dbufs[u % 4]], sems_s[u % 4],
                       add=True)

    def drain_scatter(u):
      pltpu.make_async_copy(rows[u % 3], acc.at[dbufs[u % 4]],
                            sems_s[u % 4]).wait()

    def step(i, u, first=False):
      wait_idx(u)
      issue_gather(u)
      if not first or u >= 1:
        wait_gather(u - 1)
        issue_scatter(u - 1)
      if not first or u >= 2:
        drain_scatter(u - 2)

    issue_idx(0, 0)
    issue_idx(1, 1)
    for u in range(AGG_UNROLL):
      step(u, u, first=True)
      issue_idx(u + 2, u + 2)

    def outer(g, carry):
      base = g * AGG_UNROLL
      for u in range(AGG_UNROLL):
        step(base + u, u)
        issue_idx(base + u + 2, u + 2)
      return carry

    lax.fori_loop(1, AGG_CHUNKS // AGG_UNROLL - 1, outer, None)

    last = AGG_CHUNKS - AGG_UNROLL
    for u in range(AGG_UNROLL):
      step(last + u, u)
      if u + 2 < AGG_UNROLL:
        issue_idx(last + u + 2, u + 2)
    wait_gather(AGG_UNROLL - 1)
    issue_scatter(AGG_UNROLL - 1)
    drain_scatter(AGG_UNROLL - 2)
    drain_scatter(AGG_UNROLL - 1)

  @pl.when(c == 0)
  def _():
    run(h0_hbm)

  @pl.when(c == 1)
  def _():
    run(h1_hbm)

  plsc.subcore_barrier()
  pltpu.sync_copy(acc.at[pl.ds(s * SLICE, SLICE)],
                  out_hbm.at[c, pl.ds(s * SLICE, SLICE)])


def _sc_agg(h0, h1, src_p, dst_p, zeros32):
  scratch = (
      [pltpu.VMEM((AGG_C,), jnp.int32) for _ in range(3)]
      + [pltpu.VMEM((AGG_C, HH), jnp.float32) for _ in range(3)]
      + [pltpu.SemaphoreType.DMA for _ in range(3)]
      + [pltpu.SemaphoreType.DMA for _ in range(3)]
      + [pltpu.VMEM((AGG_C,), jnp.int32) for _ in range(4)]
      + [pltpu.SemaphoreType.DMA for _ in range(4)]
      + [pltpu.VMEM_SHARED((ACC_ROWS, HH), jnp.float32)]
  )
  k = pl.kernel(
      _sc_agg_body,
      out_type=jax.ShapeDtypeStruct((2, NP, HH), jnp.float32),
      mesh=_mesh(),
      compiler_params=pltpu.CompilerParams(use_tc_tiling_on_sc=False),
      scratch_types=scratch,
  )
  return k(h0, h1, src_p, dst_p, zeros32)


def _proj_body(x_ref, w_ref, b_ref, h0_ref, h1_ref):
  y = _mm(x_ref[...], w_ref[...]) + b_ref[...]
  h0_ref[...] = y[:, :HH]
  h1_ref[...] = y[:, HH:]


def _tc_proj(x_p, w, b):
  return pl.pallas_call(
      _proj_body,
      grid=(GRID,),
      in_specs=[
          pl.BlockSpec((BLK, FEAT), lambda i: (i, 0)),
          pl.BlockSpec((FEAT, H), lambda i: (0, 0)),
          pl.BlockSpec((1, H), lambda i: (0, 0)),
      ],
      out_specs=[
          pl.BlockSpec((BLK, HH), lambda i: (i, 0)),
          pl.BlockSpec((BLK, HH), lambda i: (i, 0)),
      ],
      out_shape=[
          jax.ShapeDtypeStruct((NP, HH), jnp.float32),
          jax.ShapeDtypeStruct((NP, HH), jnp.float32),
      ],
  )(x_p, w, b)


def _layer_body(h0_ref, h1_ref, a0_ref, a1_ref, c0_ref, c1_ref,
                wl_ref, wr_ref, bb_ref, g_ref, bn_ref, *out_refs, last):
  h = jnp.concatenate([h0_ref[...], h1_ref[...]], axis=1)
  cnt = c0_ref[0] + c1_ref[0]
  cm = jnp.maximum(cnt[:, :1], 1.0)
  agg = jnp.concatenate([a0_ref[0], a1_ref[0]], axis=1) / cm
  t = _mm(agg, wl_ref[...]) + _mm(h, wr_ref[...]) + bb_ref[...]
  m = jnp.mean(t, axis=1, keepdims=True)
  d = t - m
  v = jnp.mean(d * d, axis=1, keepdims=True)
  t = d * lax.rsqrt(v + 1e-5) * g_ref[...] + bn_ref[...]
  t = jnp.maximum(t, 0.0)
  hn = h + t
  out_refs[0][...] = hn[:, :HH]
  out_refs[1][...] = hn[:, HH:]
  if last:
    out_refs[2][...] = hn
    i = pl.program_id(0)
    row = i * BLK + lax.broadcasted_iota(jnp.int32, (BLK, 1), 0)
    hm = jnp.where(row < N, hn, 0.0)
    part = jnp.sum(hm, axis=0, keepdims=True)

    @pl.when(i == 0)
    def _():
      out_refs[3][...] = jnp.zeros((1, H), jnp.float32)

    out_refs[3][...] += part


def _tc_layer(h0, h1, agg2, cnt2, wl, wr, bb, g, bn, last):
  out_specs = [
      pl.BlockSpec((BLK, HH), lambda i: (i, 0)),
      pl.BlockSpec((BLK, HH), lambda i: (i, 0)),
  ]
  out_shape = [
      jax.ShapeDtypeStruct((NP, HH), jnp.float32),
      jax.ShapeDtypeStruct((NP, HH), jnp.float32),
  ]
  if last:
    out_specs += [
        pl.BlockSpec((BLK, H), lambda i: (i, 0)),
        pl.BlockSpec((1, H), lambda i: (0, 0)),
    ]
    out_shape += [
        jax.ShapeDtypeStruct((NP, H), jnp.float32),
        jax.ShapeDtypeStruct((1, H), jnp.float32),
    ]
  return pl.pallas_call(
      functools.partial(_layer_body, last=last),
      grid=(GRID,),
      in_specs=[
          pl.BlockSpec((BLK, HH), lambda i: (i, 0)),
          pl.BlockSpec((BLK, HH), lambda i: (i, 0)),
          pl.BlockSpec((1, BLK, HH), lambda i: (0, i, 0)),
          pl.BlockSpec((1, BLK, HH), lambda i: (1, i, 0)),
          pl.BlockSpec((1, BLK, 16), lambda i: (0, i, 0)),
          pl.BlockSpec((1, BLK, 16), lambda i: (1, i, 0)),
          pl.BlockSpec((H, H), lambda i: (0, 0)),
          pl.BlockSpec((H, H), lambda i: (0, 0)),
          pl.BlockSpec((1, H), lambda i: (0, 0)),
          pl.BlockSpec((1, H), lambda i: (0, 0)),
          pl.BlockSpec((1, H), lambda i: (0, 0)),
      ],
      out_specs=out_specs,
      out_shape=out_shape,
  )(h0, h1, agg2, agg2, cnt2, cnt2, wl, wr, bb, g, bn)


def _gmlp_body(gs_ref, w1_ref, b1_ref, w2_ref, b2_ref, iw1_ref, ib1_ref,
               iw2_ref, ib2_ref, g_ref, imp_ref):
  g0 = jnp.broadcast_to(gs_ref[...] * (1.0 / N), (8, H))
  a = jnp.maximum(_mm(g0, w1_ref[...]) + b1_ref[...], 0.0)
  g = _mm(a, w2_ref[...]) + b2_ref[...]
  ib = jnp.maximum(_mm(g, iw1_ref[...]) + ib1_ref[...], 0.0)
  imp = _mm(ib, iw2_ref[...]) + ib2_ref[...]
  g_ref[...] = g[:1]
  imp_ref[...] = imp[:1]


def _tc_gmlp(gsum, p):
  return pl.pallas_call(
      _gmlp_body,
      out_shape=[
          jax.ShapeDtypeStruct((1, H), jnp.float32),
          jax.ShapeDtypeStruct((1, 3), jnp.float32),
      ],
  )(gsum, p['gp_W1'], p['gp_b1'].reshape(1, -1), p['gp_W2'],
    p['gp_b2'].reshape(1, -1), p['im_W1'], p['im_b1'].reshape(1, -1),
    p['im_W2'], p['im_b2'].reshape(1, -1))


def _heads_body(h_ref, g_ref, sw1_ref, sb1_ref, sw2_ref, sb2_ref,
                ewh_ref, ewg_ref, eb1_ref, ew2_ref, eb2_ref,
                sev_ref, sp_ref):
  h = h_ref[...]
  s1 = jnp.maximum(_mm(h, sw1_ref[...]) + sb1_ref[...], 0.0)
  sev_ref[...] = jnp.maximum(_mm(s1, sw2_ref[...]) + sb2_ref[...], 0.0)
  gg = _mm(g_ref[...], ewg_ref[...]) + eb1_ref[...]
  e1 = jnp.maximum(_mm(h, ewh_ref[...]) + gg, 0.0)
  z = _mm(e1, ew2_ref[...]) + eb2_ref[...]
  sp_ref[...] = 1.0 / (1.0 + jnp.exp(-z))


def _tc_heads(hf, g, p):
  return pl.pallas_call(
      _heads_body,
      grid=(GRID,),
      in_specs=[
          pl.BlockSpec((BLK, H), lambda i: (i, 0)),
          pl.BlockSpec((1, H), lambda i: (0, 0)),
          pl.BlockSpec((H, HH), lambda i: (0, 0)),
          pl.BlockSpec((1, HH), lambda i: (0, 0)),
          pl.BlockSpec((HH, 1), lambda i: (0, 0)),
          pl.BlockSpec((1, 1), lambda i: (0, 0)),
          pl.BlockSpec((H, HH), lambda i: (0, 0)),
          pl.BlockSpec((H, HH), lambda i: (0, 0)),
          pl.BlockSpec((1, HH), lambda i: (0, 0)),
          pl.BlockSpec((HH, 1), lambda i: (0, 0)),
          pl.BlockSpec((1, 1), lambda i: (0, 0)),
      ],
      out_specs=[
          pl.BlockSpec((BLK, 1), lambda i: (i, 0)),
          pl.BlockSpec((BLK, 1), lambda i: (i, 0)),
      ],
      out_shape=[
          jax.ShapeDtypeStruct((NP, 1), jnp.float32),
          jax.ShapeDtypeStruct((NP, 1), jnp.float32),
      ],
  )(hf, g, p['sv_W1'], p['sv_b1'].reshape(1, -1), p['sv_W2'],
    p['sv_b2'].reshape(1, -1), p['se_W1'][:H], p['se_W1'][H:],
    p['se_b1'].reshape(1, -1), p['se_W2'], p['se_b2'].reshape(1, -1))


def kernel(x, params, edge_index):
  p = params
  src = edge_index[0]
  dst = edge_index[1]

  pad_a = E_PAD - E
  src_p = jnp.concatenate(
      [src, (jnp.arange(pad_a, dtype=jnp.int32) * 127) % N])
  dst_p = jnp.concatenate(
      [dst, N + jnp.arange(pad_a, dtype=jnp.int32) % PADR])
  pad_c = E_CNT - E
  dst_c = jnp.concatenate(
      [dst, N + jnp.arange(pad_c, dtype=jnp.int32) % PADR])

  x_p = jnp.pad(x, ((0, NP - N), (0, 0)))
  zeros32 = jnp.zeros((ACC_ROWS, HH), jnp.float32)
  zeros16 = jnp.zeros((ACC_ROWS, 16), jnp.float32)
  ones16 = jnp.ones((CNT_C, 16), jnp.float32)

  cnt2 = _sc_cnt(dst_c, zeros16, ones16)
  h0, h1 = _tc_proj(x_p, p['in_W'], p['in_b'].reshape(1, -1))

  hf = gsum = None
  for i in range(3):
    agg2 = _sc_agg(h0, h1, src_p, dst_p, zeros32)
    bb = (p['l%d_bl' % i] + p['l%d_br' % i]).reshape(1, -1)
    outs = _tc_layer(h0, h1, agg2, cnt2, p['l%d_Wl' % i], p['l%d_Wr' % i],
                     bb, p['l%d_g' % i].reshape(1, -1),
                     p['l%d_bn' % i].reshape(1, -1), last=(i == 2))
    if i == 2:
      h0, h1, hf, gsum = outs
    else:
      h0, h1 = outs

  g, imp = _tc_gmlp(gsum, p)
  sev, sp = _tc_heads(hf, g, p)
  return (sev[:N], imp, sp[:N], hf[:N], g)


# speedup vs baseline: 8.2978x; 1.0612x over previous
import functools

import jax
import jax.numpy as jnp
from jax import lax
from jax.experimental import pallas as pl
from jax.experimental.pallas import tpu as pltpu
from jax.experimental.pallas import tpu_sc as plsc

N = 50000
E = 800000
FEAT = 128
H = 64
HH = 32

NP = 50176
BLK = 512
GRID = NP // BLK

ACC_ROWS = 50048
SLICE = ACC_ROWS // 16
PADR = ACC_ROWS - N

AGG_C = 264
AGG_CHUNKS = 192
AGG_UNROLL = 12
EDGES_PER_TILE = AGG_C * AGG_CHUNKS
E_PAD = 16 * EDGES_PER_TILE

E_CNT = 32 * 25600
CNT_EPW = E_CNT // 32
CNT_C = 3200
CNT_NB = CNT_EPW // CNT_C

def _mesh():
  return plsc.VectorSubcoreMesh(core_axis_name="c", subcore_axis_name="s")

DOT = functools.partial(lax.dot_general, precision=lax.Precision.DEFAULT)


def _mm(a, b):
  return DOT(a, b, (((1,), (0,)), ((), ())), preferred_element_type=jnp.float32)


def _sc_cnt_body(dst_hbm, zeros_hbm, ones_hbm, out_hbm, dbuf, ones_v, acc, sem):
  c = lax.axis_index("c")
  s = lax.axis_index("s")
  w = c * 16 + s

  pltpu.sync_copy(ones_hbm, ones_v)
  pltpu.sync_copy(zeros_hbm.at[pl.ds(s * SLICE, SLICE)],
                  acc.at[pl.ds(s * SLICE, SLICE)])
  plsc.subcore_barrier()

  def body(b, carry):
    base = w * CNT_EPW + b * CNT_C
    pltpu.sync_copy(dst_hbm.at[pl.ds(base, CNT_C)], dbuf)
    pltpu.sync_copy(ones_v, acc.at[dbuf], add=True)
    return carry

  lax.fori_loop(0, CNT_NB, body, None)
  plsc.subcore_barrier()
  pltpu.sync_copy(acc.at[pl.ds(s * SLICE, SLICE)],
                  out_hbm.at[c, pl.ds(s * SLICE, SLICE)])


def _sc_cnt(dst_cnt, zeros16, ones16):
  k = pl.kernel(
      _sc_cnt_body,
      out_type=jax.ShapeDtypeStruct((2, ACC_ROWS, 16), jnp.float32),
      mesh=_mesh(),
      compiler_params=pltpu.CompilerParams(use_tc_tiling_on_sc=False),
      scratch_types=[
          pltpu.VMEM((CNT_C,), jnp.int32),
          pltpu.VMEM((CNT_C, 16), jnp.float32),
          pltpu.VMEM_SHARED((ACC_ROWS, 16), jnp.float32),
          pltpu.SemaphoreType.DMA,
      ],
  )
  return k(dst_cnt, zeros16, ones16)


def _sc_agg_body(h0_hbm, h1_hbm, src_hbm, dst_hbm, zeros_hbm, out_hbm,
                 *scr):
  c = lax.axis_index("c")
  s = lax.axis_index("s")
  sbufs, rows, sems_i, sems_g = scr[0:3], scr[3:6], scr[6:9], scr[9:12]
  dbufs, sems_s = scr[12:16], scr[16:20]
  acc = scr[20]

  pltpu.sync_copy(zeros_hbm.at[pl.ds(s * SLICE, SLICE)],
                  acc.at[pl.ds(s * SLICE, SLICE)])
  plsc.subcore_barrier()

  def run(tab):
    ebase = s * EDGES_PER_TILE

    def issue_idx(i, u):
      nb = ebase + i * AGG_C
      pltpu.async_copy(src_hbm.at[pl.ds(nb, AGG_C)], sbufs[u % 3],
                       sems_i[u % 3])
      pltpu.async_copy(dst_hbm.at[pl.ds(nb, AGG_C)], dbufs[u % 4],
                       sems_i[u % 3])

    def wait_idx(u):
      pltpu.make_async_copy(src_hbm.at[pl.ds(0, AGG_C)], sbufs[u % 3],
                            sems_i[u % 3]).wait()
      pltpu.make_async_copy(src_hbm.at[pl.ds(0, AGG_C)], dbufs[u % 4],
                            sems_i[u % 3]).wait()

    def issue_gather(u):
      pltpu.async_copy(tab.at[sbufs[u % 3]], rows[u % 3], sems_g[u % 3])

    def wait_gather(u):
      pltpu.make_async_copy(tab.at[sbufs[u % 3]], rows[u % 3],
                            sems_g[u % 3]).wait()

    def issue_scatter(u):
      pltpu.async_copy(rows[u % 3], acc.at[dbufs[u % 4]], sems_s[u % 4],
                       add=True)

    def drain_scatter(u):
      pltpu.make_async_copy(rows[u % 3], acc.at[dbufs[u % 4]],
                            sems_s[u % 4]).wait()

    def step(i, u, first=False):
      wait_idx(u)
      issue_gather(u)
      if not first or u >= 1:
        wait_gather(u - 1)
        issue_scatter(u - 1)
      if not first or u >= 2:
        drain_scatter(u - 2)

    issue_idx(0, 0)
    issue_idx(1, 1)
    for u in range(AGG_UNROLL):
      step(u, u, first=True)
      issue_idx(u + 2, u + 2)

    def outer(g, carry):
      base = g * AGG_UNROLL
      for u in range(AGG_UNROLL):
        step(base + u, u)
        issue_idx(base + u + 2, u + 2)
      return carry

    lax.fori_loop(1, AGG_CHUNKS // AGG_UNROLL - 1, outer, None)

    last = AGG_CHUNKS - AGG_UNROLL
    for u in range(AGG_UNROLL):
      step(last + u, u)
      if u + 2 < AGG_UNROLL:
        issue_idx(last + u + 2, u + 2)
    wait_gather(AGG_UNROLL - 1)
    issue_scatter(AGG_UNROLL - 1)
    drain_scatter(AGG_UNROLL - 2)
    drain_scatter(AGG_UNROLL - 1)

  @pl.when(c == 0)
  def _():
    run(h0_hbm)

  @pl.when(c == 1)
  def _():
    run(h1_hbm)

  plsc.subcore_barrier()
  pltpu.sync_copy(acc.at[pl.ds(s * SLICE, SLICE)],
                  out_hbm.at[c, pl.ds(s * SLICE, SLICE)])


def _sc_agg(h0, h1, src_p, dst_p, zeros32):
  scratch = (
      [pltpu.VMEM((AGG_C,), jnp.int32) for _ in range(3)]
      + [pltpu.VMEM((AGG_C, HH), jnp.float32) for _ in range(3)]
      + [pltpu.SemaphoreType.DMA for _ in range(3)]
      + [pltpu.SemaphoreType.DMA for _ in range(3)]
      + [pltpu.VMEM((AGG_C,), jnp.int32) for _ in range(4)]
      + [pltpu.SemaphoreType.DMA for _ in range(4)]
      + [pltpu.VMEM_SHARED((ACC_ROWS, HH), jnp.float32)]
  )
  k = pl.kernel(
      _sc_agg_body,
      out_type=jax.ShapeDtypeStruct((2, ACC_ROWS, HH), jnp.float32),
      mesh=_mesh(),
      compiler_params=pltpu.CompilerParams(use_tc_tiling_on_sc=False),
      scratch_types=scratch,
  )
  return k(h0, h1, src_p, dst_p, zeros32)


def _proj_body(x_ref, w_ref, b_ref, h0_ref, h1_ref):
  y = _mm(x_ref[...], w_ref[...]) + b_ref[...]
  h0_ref[...] = y[:, :HH]
  h1_ref[...] = y[:, HH:]


def _tc_proj(xx, w, b):
  return pl.pallas_call(
      _proj_body,
      grid=(GRID,),
      in_specs=[
          pl.BlockSpec((BLK, FEAT), lambda i: (i, 0)),
          pl.BlockSpec((FEAT, H), lambda i: (0, 0)),
          pl.BlockSpec((1, H), lambda i: (0, 0)),
      ],
      out_specs=[
          pl.BlockSpec((BLK, HH), lambda i: (i, 0)),
          pl.BlockSpec((BLK, HH), lambda i: (i, 0)),
      ],
      out_shape=[
          jax.ShapeDtypeStruct((N, HH), jnp.float32),
          jax.ShapeDtypeStruct((N, HH), jnp.float32),
      ],
  )(xx, w, b)


def _layer_body(h0_ref, h1_ref, a0_ref, a1_ref, c0_ref, c1_ref,
                wl_ref, wr_ref, bb_ref, g_ref, bn_ref, *out_refs, last):
  h = jnp.concatenate([h0_ref[...], h1_ref[...]], axis=1)
  cnt = c0_ref[0] + c1_ref[0]
  cm = jnp.maximum(cnt[:, :1], 1.0)
  agg = jnp.concatenate([a0_ref[0], a1_ref[0]], axis=1) / cm
  t = _mm(agg, wl_ref[...]) + _mm(h, wr_ref[...]) + bb_ref[...]
  m = jnp.mean(t, axis=1, keepdims=True)
  d = t - m
  v = jnp.mean(d * d, axis=1, keepdims=True)
  t = d * lax.rsqrt(v + 1e-5) * g_ref[...] + bn_ref[...]
  t = jnp.maximum(t, 0.0)
  hn = h + t
  if last:
    out_refs[0][...] = hn
    i = pl.program_id(0)
    row = i * BLK + lax.broadcasted_iota(jnp.int32, (BLK, 1), 0)
    hm = jnp.where(row < N, hn, 0.0)
    part = jnp.sum(hm, axis=0, keepdims=True)

    @pl.when(i == 0)
    def _():
      out_refs[1][...] = jnp.zeros((1, H), jnp.float32)

    out_refs[1][...] += part
  else:
    out_refs[0][...] = hn[:, :HH]
    out_refs[1][...] = hn[:, HH:]


def _tc_layer(h0, h1, agg2, cnt2, wl, wr, bb, g, bn, last):
  if last:
    out_specs = [
        pl.BlockSpec((BLK, H), lambda i: (i, 0)),
        pl.BlockSpec((1, H), lambda i: (0, 0)),
    ]
    out_shape = [
        jax.ShapeDtypeStruct((N, H), jnp.float32),
        jax.ShapeDtypeStruct((1, H), jnp.float32),
    ]
  else:
    out_specs = [
        pl.BlockSpec((BLK, HH), lambda i: (i, 0)),
        pl.BlockSpec((BLK, HH), lambda i: (i, 0)),
    ]
    out_shape = [
        jax.ShapeDtypeStruct((N, HH), jnp.float32),
        jax.ShapeDtypeStruct((N, HH), jnp.float32),
    ]
  return pl.pallas_call(
      functools.partial(_layer_body, last=last),
      grid=(GRID,),
      in_specs=[
          pl.BlockSpec((BLK, HH), lambda i: (i, 0)),
          pl.BlockSpec((BLK, HH), lambda i: (i, 0)),
          pl.BlockSpec((1, BLK, HH), lambda i: (0, i, 0)),
          pl.BlockSpec((1, BLK, HH), lambda i: (1, i, 0)),
          pl.BlockSpec((1, BLK, 16), lambda i: (0, i, 0)),
          pl.BlockSpec((1, BLK, 16), lambda i: (1, i, 0)),
          pl.BlockSpec((H, H), lambda i: (0, 0)),
          pl.BlockSpec((H, H), lambda i: (0, 0)),
          pl.BlockSpec((1, H), lambda i: (0, 0)),
          pl.BlockSpec((1, H), lambda i: (0, 0)),
          pl.BlockSpec((1, H), lambda i: (0, 0)),
      ],
      out_specs=out_specs,
      out_shape=out_shape,
  )(h0, h1, agg2, agg2, cnt2, cnt2, wl, wr, bb, g, bn)


def _gmlp_body(gs_ref, w1_ref, b1_ref, w2_ref, b2_ref, iw1_ref, ib1_ref,
               iw2_ref, ib2_ref, g_ref, imp_ref):
  g0 = jnp.broadcast_to(gs_ref[...] * (1.0 / N), (8, H))
  a = jnp.maximum(_mm(g0, w1_ref[...]) + b1_ref[...], 0.0)
  g = _mm(a, w2_ref[...]) + b2_ref[...]
  ib = jnp.maximum(_mm(g, iw1_ref[...]) + ib1_ref[...], 0.0)
  imp = _mm(ib, iw2_ref[...]) + ib2_ref[...]
  g_ref[...] = g[:1]
  imp_ref[...] = imp[:1]


def _tc_gmlp(gsum, p):
  return pl.pallas_call(
      _gmlp_body,
      out_shape=[
          jax.ShapeDtypeStruct((1, H), jnp.float32),
          jax.ShapeDtypeStruct((1, 3), jnp.float32),
      ],
  )(gsum, p['gp_W1'], p['gp_b1'].reshape(1, -1), p['gp_W2'],
    p['gp_b2'].reshape(1, -1), p['im_W1'], p['im_b1'].reshape(1, -1),
    p['im_W2'], p['im_b2'].reshape(1, -1))


def _heads_body(h_ref, g_ref, sw1_ref, sb1_ref, sw2_ref, sb2_ref,
                ewh_ref, ewg_ref, eb1_ref, ew2_ref, eb2_ref,
                sev_ref, sp_ref):
  h = h_ref[...]
  s1 = jnp.maximum(_mm(h, sw1_ref[...]) + sb1_ref[...], 0.0)
  sev_ref[...] = jnp.maximum(_mm(s1, sw2_ref[...]) + sb2_ref[...], 0.0)
  gg = _mm(g_ref[...], ewg_ref[...]) + eb1_ref[...]
  e1 = jnp.maximum(_mm(h, ewh_ref[...]) + gg, 0.0)
  z = _mm(e1, ew2_ref[...]) + eb2_ref[...]
  sp_ref[...] = 1.0 / (1.0 + jnp.exp(-z))


def _tc_heads(hf, g, p):
  return pl.pallas_call(
      _heads_body,
      grid=(GRID,),
      in_specs=[
          pl.BlockSpec((BLK, H), lambda i: (i, 0)),
          pl.BlockSpec((1, H), lambda i: (0, 0)),
          pl.BlockSpec((H, HH), lambda i: (0, 0)),
          pl.BlockSpec((1, HH), lambda i: (0, 0)),
          pl.BlockSpec((HH, 1), lambda i: (0, 0)),
          pl.BlockSpec((1, 1), lambda i: (0, 0)),
          pl.BlockSpec((H, HH), lambda i: (0, 0)),
          pl.BlockSpec((H, HH), lambda i: (0, 0)),
          pl.BlockSpec((1, HH), lambda i: (0, 0)),
          pl.BlockSpec((HH, 1), lambda i: (0, 0)),
          pl.BlockSpec((1, 1), lambda i: (0, 0)),
      ],
      out_specs=[
          pl.BlockSpec((BLK, 1), lambda i: (i, 0)),
          pl.BlockSpec((BLK, 1), lambda i: (i, 0)),
      ],
      out_shape=[
          jax.ShapeDtypeStruct((N, 1), jnp.float32),
          jax.ShapeDtypeStruct((N, 1), jnp.float32),
      ],
  )(hf, g, p['sv_W1'], p['sv_b1'].reshape(1, -1), p['sv_W2'],
    p['sv_b2'].reshape(1, -1), p['se_W1'][:H], p['se_W1'][H:],
    p['se_b1'].reshape(1, -1), p['se_W2'], p['se_b2'].reshape(1, -1))


def kernel(x, params, edge_index):
  p = params
  src = edge_index[0]
  dst = edge_index[1]

  pad_a = E_PAD - E
  src_p = jnp.concatenate(
      [src, (jnp.arange(pad_a, dtype=jnp.int32) * 127) % N])
  dst_p = jnp.concatenate(
      [dst, N + jnp.arange(pad_a, dtype=jnp.int32) % PADR])
  pad_c = E_CNT - E
  dst_c = jnp.concatenate(
      [dst, N + jnp.arange(pad_c, dtype=jnp.int32) % PADR])

  zeros32 = jnp.zeros((ACC_ROWS, HH), jnp.float32)
  zeros16 = jnp.zeros((ACC_ROWS, 16), jnp.float32)
  ones16 = jnp.ones((CNT_C, 16), jnp.float32)

  cnt2 = _sc_cnt(dst_c, zeros16, ones16)
  h0, h1 = _tc_proj(x, p['in_W'], p['in_b'].reshape(1, -1))

  hf = gsum = None
  for i in range(3):
    agg2 = _sc_agg(h0, h1, src_p, dst_p, zeros32)
    bb = (p['l%d_bl' % i] + p['l%d_br' % i]).reshape(1, -1)
    outs = _tc_layer(h0, h1, agg2, cnt2, p['l%d_Wl' % i], p['l%d_Wr' % i],
                     bb, p['l%d_g' % i].reshape(1, -1),
                     p['l%d_bn' % i].reshape(1, -1), last=(i == 2))
    if i == 2:
      hf, gsum = outs
    else:
      h0, h1 = outs

  g, imp = _tc_gmlp(gsum, p)
  sev, sp = _tc_heads(hf, g, p)
  return (sev, imp, sp, hf, g)


# speedup vs baseline: 8.8703x; 1.0690x over previous
import functools

import jax
import jax.numpy as jnp
from jax import lax
from jax.experimental import pallas as pl
from jax.experimental.pallas import tpu as pltpu
from jax.experimental.pallas import tpu_sc as plsc

N = 50000
E = 800000
FEAT = 128
H = 64
HH = 32

NP = 50176
BLK = 512
GRID = NP // BLK

ACC_ROWS = 50048
SLICE = ACC_ROWS // 16
PADR = ACC_ROWS - N
PACKED = ACC_ROWS // 4
CPACK = ACC_ROWS // 8
PBLK = BLK // 4
CBLK = BLK // 8

AGG_C = 264
AGG_CHUNKS = 192
AGG_UNROLL = 12
EDGES_PER_TILE = AGG_C * AGG_CHUNKS
E_PAD = 16 * EDGES_PER_TILE

E_CNT = 32 * 25600
CNT_EPW = E_CNT // 32
CNT_C = 3200
CNT_NB = CNT_EPW // CNT_C

def _mesh():
  return plsc.VectorSubcoreMesh(core_axis_name="c", subcore_axis_name="s")

DOT = functools.partial(lax.dot_general, precision=lax.Precision.DEFAULT)


def _mm(a, b):
  return DOT(a, b, (((1,), (0,)), ((), ())), preferred_element_type=jnp.float32)


def _sc_cnt_body(dst_hbm, zeros_hbm, ones_hbm, out_hbm, dbuf, ones_v, acc, sem):
  c = lax.axis_index("c")
  s = lax.axis_index("s")
  w = c * 16 + s

  pltpu.sync_copy(ones_hbm, ones_v)
  pltpu.sync_copy(zeros_hbm.at[pl.ds(s * SLICE, SLICE)],
                  acc.at[pl.ds(s * SLICE, SLICE)])
  plsc.subcore_barrier()

  def body(b, carry):
    base = w * CNT_EPW + b * CNT_C
    pltpu.sync_copy(dst_hbm.at[pl.ds(base, CNT_C)], dbuf)
    pltpu.sync_copy(ones_v, acc.at[dbuf], add=True)
    return carry

  lax.fori_loop(0, CNT_NB, body, None)
  plsc.subcore_barrier()
  pltpu.sync_copy(acc.at[pl.ds(s * SLICE, SLICE)],
                  out_hbm.at[pl.ds(s * SLICE, SLICE), pl.ds(c * 16, 16)])


def _sc_cnt(dst_cnt, zeros16, ones16):
  k = pl.kernel(
      _sc_cnt_body,
      out_type=jax.ShapeDtypeStruct((ACC_ROWS, 128), jnp.float32),
      mesh=_mesh(),
      compiler_params=pltpu.CompilerParams(use_tc_tiling_on_sc=False),
      scratch_types=[
          pltpu.VMEM((CNT_C,), jnp.int32),
          pltpu.VMEM((CNT_C, 16), jnp.float32),
          pltpu.VMEM_SHARED((ACC_ROWS, 16), jnp.float32),
          pltpu.SemaphoreType.DMA,
      ],
  )
  return k(dst_cnt, zeros16, ones16)


def _sc_agg_body(h0_hbm, h1_hbm, src_hbm, dst_hbm, zeros_hbm, out_hbm,
                 *scr):
  c = lax.axis_index("c")
  s = lax.axis_index("s")
  sbufs, rows, sems_i, sems_g = scr[0:3], scr[3:6], scr[6:9], scr[9:12]
  dbufs, sems_s = scr[12:16], scr[16:20]
  acc = scr[20]

  pltpu.sync_copy(zeros_hbm.at[pl.ds(s * SLICE, SLICE)],
                  acc.at[pl.ds(s * SLICE, SLICE)])
  plsc.subcore_barrier()

  def run(tab):
    ebase = s * EDGES_PER_TILE

    def issue_idx(i, u):
      nb = ebase + i * AGG_C
      pltpu.async_copy(src_hbm.at[pl.ds(nb, AGG_C)], sbufs[u % 3],
                       sems_i[u % 3])
      pltpu.async_copy(dst_hbm.at[pl.ds(nb, AGG_C)], dbufs[u % 4],
                       sems_i[u % 3])

    def wait_idx(u):
      pltpu.make_async_copy(src_hbm.at[pl.ds(0, AGG_C)], sbufs[u % 3],
                            sems_i[u % 3]).wait()
      pltpu.make_async_copy(src_hbm.at[pl.ds(0, AGG_C)], dbufs[u % 4],
                            sems_i[u % 3]).wait()

    def issue_gather(u):
      pltpu.async_copy(tab.at[sbufs[u % 3]], rows[u % 3], sems_g[u % 3])

    def wait_gather(u):
      pltpu.make_async_copy(tab.at[sbufs[u % 3]], rows[u % 3],
                            sems_g[u % 3]).wait()

    def issue_scatter(u):
      pltpu.async_copy(rows[u % 3], acc.at[dbufs[u % 4]], sems_s[u % 4],
                       add=True)

    def drain_scatter(u):
      pltpu.make_async_copy(rows[u % 3], acc.at[dbufs[u % 4]],
                            sems_s[u % 4]).wait()

    def step(i, u, first=False):
      wait_idx(u)
      issue_gather(u)
      if not first or u >= 1:
        wait_gather(u - 1)
        issue_scatter(u - 1)
      if not first or u >= 2:
        drain_scatter(u - 2)

    issue_idx(0, 0)
    issue_idx(1, 1)
    for u in range(AGG_UNROLL):
      step(u, u, first=True)
      issue_idx(u + 2, u + 2)

    def outer(g, carry):
      base = g * AGG_UNROLL
      for u in range(AGG_UNROLL):
        step(base + u, u)
        issue_idx(base + u + 2, u + 2)
      return carry

    lax.fori_loop(1, AGG_CHUNKS // AGG_UNROLL - 1, outer, None)

    last = AGG_CHUNKS - AGG_UNROLL
    for u in range(AGG_UNROLL):
      step(last + u, u)
      if u + 2 < AGG_UNROLL:
        issue_idx(last + u + 2, u + 2)
    wait_gather(AGG_UNROLL - 1)
    issue_scatter(AGG_UNROLL - 1)
    drain_scatter(AGG_UNROLL - 2)
    drain_scatter(AGG_UNROLL - 1)

  @pl.when(c == 0)
  def _():
    run(h0_hbm)

  @pl.when(c == 1)
  def _():
    run(h1_hbm)

  plsc.subcore_barrier()
  pltpu.sync_copy(acc.at[pl.ds(s * SLICE, SLICE)],
                  out_hbm.at[pl.ds(s * SLICE, SLICE), pl.ds(c * HH, HH)])


def _sc_agg(h0, h1, src_p, dst_p, zeros32):
  scratch = (
      [pltpu.VMEM((AGG_C,), jnp.int32) for _ in range(3)]
      + [pltpu.VMEM((AGG_C, HH), jnp.float32) for _ in range(3)]
      + [pltpu.SemaphoreType.DMA for _ in range(3)]
      + [pltpu.SemaphoreType.DMA for _ in range(3)]
      + [pltpu.VMEM((AGG_C,), jnp.int32) for _ in range(4)]
      + [pltpu.SemaphoreType.DMA for _ in range(4)]
      + [pltpu.VMEM_SHARED((ACC_ROWS, HH), jnp.float32)]
  )
  k = pl.kernel(
      _sc_agg_body,
      out_type=jax.ShapeDtypeStruct((ACC_ROWS, 128), jnp.float32),
      mesh=_mesh(),
      compiler_params=pltpu.CompilerParams(use_tc_tiling_on_sc=False),
      scratch_types=scratch,
  )
  return k(h0, h1, src_p, dst_p, zeros32)


def _proj_body(x_ref, w_ref, b_ref, h0_ref, h1_ref):
  y = _mm(x_ref[...], w_ref[...]) + b_ref[...]
  h0_ref[...] = y[:, :HH]
  h1_ref[...] = y[:, HH:]


def _tc_proj(xx, w, b):
  return pl.pallas_call(
      _proj_body,
      grid=(GRID,),
      in_specs=[
          pl.BlockSpec((BLK, FEAT), lambda i: (i, 0)),
          pl.BlockSpec((FEAT, H), lambda i: (0, 0)),
          pl.BlockSpec((1, H), lambda i: (0, 0)),
      ],
      out_specs=[
          pl.BlockSpec((BLK, HH), lambda i: (i, 0)),
          pl.BlockSpec((BLK, HH), lambda i: (i, 0)),
      ],
      out_shape=[
          jax.ShapeDtypeStruct((N, HH), jnp.float32),
          jax.ShapeDtypeStruct((N, HH), jnp.float32),
      ],
  )(xx, w, b)


def _layer_body(h0_ref, h1_ref, a_ref, c_ref,
                wl_ref, wr_ref, bb_ref, g_ref, bn_ref, *out_refs, last):
  h = jnp.concatenate([h0_ref[...], h1_ref[...]], axis=1)
  cb = c_ref[...]
  cm = jnp.maximum(cb[:, :1] + cb[:, 16:17], 1.0)
  agg = a_ref[...][:, :H] / cm
  t = _mm(agg, wl_ref[...]) + _mm(h, wr_ref[...]) + bb_ref[...]
  m = jnp.mean(t, axis=1, keepdims=True)
  d = t - m
  v = jnp.mean(d * d, axis=1, keepdims=True)
  t = d * lax.rsqrt(v + 1e-5) * g_ref[...] + bn_ref[...]
  t = jnp.maximum(t, 0.0)
  hn = h + t
  if last:
    out_refs[0][...] = hn
    i = pl.program_id(0)
    row = i * BLK + lax.broadcasted_iota(jnp.int32, (BLK, 1), 0)
    hm = jnp.where(row < N, hn, 0.0)
    part = jnp.sum(hm, axis=0, keepdims=True)

    @pl.when(i == 0)
    def _():
      out_refs[1][...] = jnp.zeros((1, H), jnp.float32)

    out_refs[1][...] += part
  else:
    out_refs[0][...] = hn[:, :HH]
    out_refs[1][...] = hn[:, HH:]


def _tc_layer(h0, h1, agg2, cnt2, wl, wr, bb, g, bn, last):
  if last:
    out_specs = [
        pl.BlockSpec((BLK, H), lambda i: (i, 0)),
        pl.BlockSpec((1, H), lambda i: (0, 0)),
    ]
    out_shape = [
        jax.ShapeDtypeStruct((N, H), jnp.float32),
        jax.ShapeDtypeStruct((1, H), jnp.float32),
    ]
  else:
    out_specs = [
        pl.BlockSpec((BLK, HH), lambda i: (i, 0)),
        pl.BlockSpec((BLK, HH), lambda i: (i, 0)),
    ]
    out_shape = [
        jax.ShapeDtypeStruct((N, HH), jnp.float32),
        jax.ShapeDtypeStruct((N, HH), jnp.float32),
    ]
  return pl.pallas_call(
      functools.partial(_layer_body, last=last),
      grid=(GRID,),
      in_specs=[
          pl.BlockSpec((BLK, HH), lambda i: (i, 0)),
          pl.BlockSpec((BLK, HH), lambda i: (i, 0)),
          pl.BlockSpec((BLK, 128), lambda i: (i, 0)),
          pl.BlockSpec((BLK, 128), lambda i: (i, 0)),
          pl.BlockSpec((H, H), lambda i: (0, 0)),
          pl.BlockSpec((H, H), lambda i: (0, 0)),
          pl.BlockSpec((1, H), lambda i: (0, 0)),
          pl.BlockSpec((1, H), lambda i: (0, 0)),
          pl.BlockSpec((1, H), lambda i: (0, 0)),
      ],
      out_specs=out_specs,
      out_shape=out_shape,
  )(h0, h1, agg2, cnt2, wl, wr, bb, g, bn)


def _gmlp_body(gs_ref, w1_ref, b1_ref, w2_ref, b2_ref, iw1_ref, ib1_ref,
               iw2_ref, ib2_ref, g_ref, imp_ref):
  g0 = jnp.broadcast_to(gs_ref[...] * (1.0 / N), (8, H))
  a = jnp.maximum(_mm(g0, w1_ref[...]) + b1_ref[...], 0.0)
  g = _mm(a, w2_ref[...]) + b2_ref[...]
  ib = jnp.maximum(_mm(g, iw1_ref[...]) + ib1_ref[...], 0.0)
  imp = _mm(ib, iw2_ref[...]) + ib2_ref[...]
  g_ref[...] = g[:1]
  imp_ref[...] = imp[:1]


def _tc_gmlp(gsum, p):
  return pl.pallas_call(
      _gmlp_body,
      out_shape=[
          jax.ShapeDtypeStruct((1, H), jnp.float32),
          jax.ShapeDtypeStruct((1, 3), jnp.float32),
      ],
  )(gsum, p['gp_W1'], p['gp_b1'].reshape(1, -1), p['gp_W2'],
    p['gp_b2'].reshape(1, -1), p['im_W1'], p['im_b1'].reshape(1, -1),
    p['im_W2'], p['im_b2'].reshape(1, -1))


def _heads_body(h_ref, g_ref, sw1_ref, sb1_ref, sw2_ref, sb2_ref,
                ewh_ref, ewg_ref, eb1_ref, ew2_ref, eb2_ref,
                sev_ref, sp_ref):
  h = h_ref[...]
  s1 = jnp.maximum(_mm(h, sw1_ref[...]) + sb1_ref[...], 0.0)
  sev_ref[...] = jnp.maximum(_mm(s1, sw2_ref[...]) + sb2_ref[...], 0.0)
  gg = _mm(g_ref[...], ewg_ref[...]) + eb1_ref[...]
  e1 = jnp.maximum(_mm(h, ewh_ref[...]) + gg, 0.0)
  z = _mm(e1, ew2_ref[...]) + eb2_ref[...]
  sp_ref[...] = 1.0 / (1.0 + jnp.exp(-z))


def _tc_heads(hf, g, p):
  return pl.pallas_call(
      _heads_body,
      grid=(GRID,),
      in_specs=[
          pl.BlockSpec((BLK, H), lambda i: (i, 0)),
          pl.BlockSpec((1, H), lambda i: (0, 0)),
          pl.BlockSpec((H, HH), lambda i: (0, 0)),
          pl.BlockSpec((1, HH), lambda i: (0, 0)),
          pl.BlockSpec((HH, 1), lambda i: (0, 0)),
          pl.BlockSpec((1, 1), lambda i: (0, 0)),
          pl.BlockSpec((H, HH), lambda i: (0, 0)),
          pl.BlockSpec((H, HH), lambda i: (0, 0)),
          pl.BlockSpec((1, HH), lambda i: (0, 0)),
          pl.BlockSpec((HH, 1), lambda i: (0, 0)),
          pl.BlockSpec((1, 1), lambda i: (0, 0)),
      ],
      out_specs=[
          pl.BlockSpec((BLK, 1), lambda i: (i, 0)),
          pl.BlockSpec((BLK, 1), lambda i: (i, 0)),
      ],
      out_shape=[
          jax.ShapeDtypeStruct((N, 1), jnp.float32),
          jax.ShapeDtypeStruct((N, 1), jnp.float32),
      ],
  )(hf, g, p['sv_W1'], p['sv_b1'].reshape(1, -1), p['sv_W2'],
    p['sv_b2'].reshape(1, -1), p['se_W1'][:H], p['se_W1'][H:],
    p['se_b1'].reshape(1, -1), p['se_W2'], p['se_b2'].reshape(1, -1))


def kernel(x, params, edge_index):
  p = params
  src = edge_index[0]
  dst = edge_index[1]

  pad_a = E_PAD - E
  src_p = jnp.concatenate(
      [src, (jnp.arange(pad_a, dtype=jnp.int32) * 127) % N])
  dst_p = jnp.concatenate(
      [dst, N + jnp.arange(pad_a, dtype=jnp.int32) % PADR])
  pad_c = E_CNT - E
  dst_c = jnp.concatenate(
      [dst, N + jnp.arange(pad_c, dtype=jnp.int32) % PADR])

  zeros32 = jnp.zeros((ACC_ROWS, HH), jnp.float32)
  zeros16 = jnp.zeros((ACC_ROWS, 16), jnp.float32)
  ones16 = jnp.ones((CNT_C, 16), jnp.float32)

  cnt2 = _sc_cnt(dst_c, zeros16, ones16)
  h0, h1 = _tc_proj(x, p['in_W'], p['in_b'].reshape(1, -1))

  hf = gsum = None
  for i in range(3):
    agg2 = _sc_agg(h0, h1, src_p, dst_p, zeros32)
    bb = (p['l%d_bl' % i] + p['l%d_br' % i]).reshape(1, -1)
    outs = _tc_layer(h0, h1, agg2, cnt2, p['l%d_Wl' % i], p['l%d_Wr' % i],
                     bb, p['l%d_g' % i].reshape(1, -1),
                     p['l%d_bn' % i].reshape(1, -1), last=(i == 2))
    if i == 2:
      hf, gsum = outs
    else:
      h0, h1 = outs

  g, imp = _tc_gmlp(gsum, p)
  sev, sp = _tc_heads(hf, g, p)
  return (sev, imp, sp, hf, g)


# speedup vs baseline: 9.1475x; 1.0312x over previous
import functools

import jax
import jax.numpy as jnp
from jax import lax
from jax.experimental import pallas as pl
from jax.experimental.pallas import tpu as pltpu
from jax.experimental.pallas import tpu_sc as plsc

N = 50000
E = 800000
FEAT = 128
H = 64
HH = 32

NP = 50176
BLK = 512
GRID = NP // BLK

ACC_ROWS = 50048
SLICE = ACC_ROWS // 16
PADR = ACC_ROWS - N
PACKED = ACC_ROWS // 4
CPACK = ACC_ROWS // 8
PBLK = BLK // 4
CBLK = BLK // 8

AGG_C = 264
AGG_CHUNKS = 192
AGG_UNROLL = 12
EDGES_PER_TILE = AGG_C * AGG_CHUNKS
E_PAD = 16 * EDGES_PER_TILE

E_CNT = 32 * 25600
CNT_EPW = E_CNT // 32
CNT_C = 3200
CNT_NB = CNT_EPW // CNT_C

def _mesh():
  return plsc.VectorSubcoreMesh(core_axis_name="c", subcore_axis_name="s")

DOT = functools.partial(lax.dot_general, precision=lax.Precision.DEFAULT)


def _mm(a, b):
  return DOT(a, b, (((1,), (0,)), ((), ())), preferred_element_type=jnp.float32)


def _sc_cnt_body(dst_hbm, zeros_hbm, ones_hbm, out_hbm, dbuf, ones_v, acc, sem):
  c = lax.axis_index("c")
  s = lax.axis_index("s")
  w = c * 16 + s

  pltpu.sync_copy(ones_hbm, ones_v)
  pltpu.sync_copy(zeros_hbm.at[pl.ds(s * SLICE, SLICE)],
                  acc.at[pl.ds(s * SLICE, SLICE)])
  plsc.subcore_barrier()

  def body(b, carry):
    base = w * CNT_EPW + b * CNT_C
    pltpu.sync_copy(dst_hbm.at[pl.ds(base, CNT_C)], dbuf)
    pltpu.sync_copy(ones_v, acc.at[dbuf], add=True)
    return carry

  lax.fori_loop(0, CNT_NB, body, None)
  plsc.subcore_barrier()
  pltpu.sync_copy(acc.at[pl.ds(s * SLICE, SLICE)],
                  out_hbm.at[c, pl.ds(s * SLICE, SLICE)])


def _sc_cnt(dst_cnt, zeros16, ones16):
  k = pl.kernel(
      _sc_cnt_body,
      out_type=jax.ShapeDtypeStruct((2, ACC_ROWS, 16), jnp.float32),
      mesh=_mesh(),
      compiler_params=pltpu.CompilerParams(use_tc_tiling_on_sc=False),
      scratch_types=[
          pltpu.VMEM((CNT_C,), jnp.int32),
          pltpu.VMEM((CNT_C, 16), jnp.float32),
          pltpu.VMEM_SHARED((ACC_ROWS, 16), jnp.float32),
          pltpu.SemaphoreType.DMA,
      ],
  )
  return k(dst_cnt, zeros16, ones16)


def _sc_agg_body(h0_hbm, h1_hbm, src_hbm, dst_hbm, zeros_hbm, out_hbm,
                 *scr):
  c = lax.axis_index("c")
  s = lax.axis_index("s")
  sbufs, rows, sems_i, sems_g = scr[0:3], scr[3:6], scr[6:9], scr[9:12]
  dbufs, sems_s = scr[12:16], scr[16:20]
  acc = scr[20]

  pltpu.sync_copy(zeros_hbm.at[pl.ds(s * SLICE, SLICE)],
                  acc.at[pl.ds(s * SLICE, SLICE)])
  plsc.subcore_barrier()

  def run(tab):
    ebase = s * EDGES_PER_TILE

    def issue_idx(i, u):
      nb = ebase + i * AGG_C
      pltpu.async_copy(src_hbm.at[pl.ds(nb, AGG_C)], sbufs[u % 3],
                       sems_i[u % 3])
      pltpu.async_copy(dst_hbm.at[pl.ds(nb, AGG_C)], dbufs[u % 4],
                       sems_i[u % 3])

    def wait_idx(u):
      pltpu.make_async_copy(src_hbm.at[pl.ds(0, AGG_C)], sbufs[u % 3],
                            sems_i[u % 3]).wait()
      pltpu.make_async_copy(src_hbm.at[pl.ds(0, AGG_C)], dbufs[u % 4],
                            sems_i[u % 3]).wait()

    def issue_gather(u):
      pltpu.async_copy(tab.at[sbufs[u % 3]], rows[u % 3], sems_g[u % 3])

    def wait_gather(u):
      pltpu.make_async_copy(tab.at[sbufs[u % 3]], rows[u % 3],
                            sems_g[u % 3]).wait()

    def issue_scatter(u):
      pltpu.async_copy(rows[u % 3], acc.at[dbufs[u % 4]], sems_s[u % 4],
                       add=True)

    def drain_scatter(u):
      pltpu.make_async_copy(rows[u % 3], acc.at[dbufs[u % 4]],
                            sems_s[u % 4]).wait()

    def step(i, u, first=False):
      wait_idx(u)
      issue_gather(u)
      if not first or u >= 1:
        wait_gather(u - 1)
        issue_scatter(u - 1)
      if not first or u >= 2:
        drain_scatter(u - 2)

    issue_idx(0, 0)
    issue_idx(1, 1)
    for u in range(AGG_UNROLL):
      step(u, u, first=True)
      issue_idx(u + 2, u + 2)

    def outer(g, carry):
      base = g * AGG_UNROLL
      for u in range(AGG_UNROLL):
        step(base + u, u)
        issue_idx(base + u + 2, u + 2)
      return carry

    lax.fori_loop(1, AGG_CHUNKS // AGG_UNROLL - 1, outer, None)

    last = AGG_CHUNKS - AGG_UNROLL
    for u in range(AGG_UNROLL):
      step(last + u, u)
      if u + 2 < AGG_UNROLL:
        issue_idx(last + u + 2, u + 2)
    wait_gather(AGG_UNROLL - 1)
    issue_scatter(AGG_UNROLL - 1)
    drain_scatter(AGG_UNROLL - 2)
    drain_scatter(AGG_UNROLL - 1)

  @pl.when(c == 0)
  def _():
    run(h0_hbm)

  @pl.when(c == 1)
  def _():
    run(h1_hbm)

  plsc.subcore_barrier()
  pltpu.sync_copy(acc.at[pl.ds(s * SLICE, SLICE)],
                  out_hbm.at[pl.ds(s * SLICE, SLICE), pl.ds(c * HH, HH)])


def _sc_agg(h0, h1, src_p, dst_p, zeros32):
  scratch = (
      [pltpu.VMEM((AGG_C,), jnp.int32) for _ in range(3)]
      + [pltpu.VMEM((AGG_C, HH), jnp.float32) for _ in range(3)]
      + [pltpu.SemaphoreType.DMA for _ in range(3)]
      + [pltpu.SemaphoreType.DMA for _ in range(3)]
      + [pltpu.VMEM((AGG_C,), jnp.int32) for _ in range(4)]
      + [pltpu.SemaphoreType.DMA for _ in range(4)]
      + [pltpu.VMEM_SHARED((ACC_ROWS, HH), jnp.float32)]
  )
  k = pl.kernel(
      _sc_agg_body,
      out_type=jax.ShapeDtypeStruct((ACC_ROWS, 128), jnp.float32),
      mesh=_mesh(),
      compiler_params=pltpu.CompilerParams(use_tc_tiling_on_sc=False),
      scratch_types=scratch,
  )
  return k(h0, h1, src_p, dst_p, zeros32)


def _proj_body(x_ref, w_ref, b_ref, h0_ref, h1_ref):
  y = _mm(x_ref[...], w_ref[...]) + b_ref[...]
  h0_ref[...] = y[:, :HH]
  h1_ref[...] = y[:, HH:]


def _tc_proj(xx, w, b):
  return pl.pallas_call(
      _proj_body,
      grid=(GRID,),
      in_specs=[
          pl.BlockSpec((BLK, FEAT), lambda i: (i, 0)),
          pl.BlockSpec((FEAT, H), lambda i: (0, 0)),
          pl.BlockSpec((1, H), lambda i: (0, 0)),
      ],
      out_specs=[
          pl.BlockSpec((BLK, HH), lambda i: (i, 0)),
          pl.BlockSpec((BLK, HH), lambda i: (i, 0)),
      ],
      out_shape=[
          jax.ShapeDtypeStruct((N, HH), jnp.float32),
          jax.ShapeDtypeStruct((N, HH), jnp.float32),
      ],
  )(xx, w, b)


def _layer_body(h0_ref, h1_ref, a_ref, c0_ref, c1_ref,
                wl_ref, wr_ref, bb_ref, g_ref, bn_ref, *out_refs, last):
  h = jnp.concatenate([h0_ref[...], h1_ref[...]], axis=1)
  cnt = c0_ref[0] + c1_ref[0]
  cm = jnp.maximum(cnt[:, :1], 1.0)
  agg = a_ref[...][:, :H] / cm
  t = _mm(agg, wl_ref[...]) + _mm(h, wr_ref[...]) + bb_ref[...]
  m = jnp.mean(t, axis=1, keepdims=True)
  d = t - m
  v = jnp.mean(d * d, axis=1, keepdims=True)
  t = d * lax.rsqrt(v + 1e-5) * g_ref[...] + bn_ref[...]
  t = jnp.maximum(t, 0.0)
  hn = h + t
  if last:
    out_refs[0][...] = hn
    i = pl.program_id(0)
    row = i * BLK + lax.broadcasted_iota(jnp.int32, (BLK, 1), 0)
    hm = jnp.where(row < N, hn, 0.0)
    part = jnp.sum(hm, axis=0, keepdims=True)

    @pl.when(i == 0)
    def _():
      out_refs[1][...] = jnp.zeros((1, H), jnp.float32)

    out_refs[1][...] += part
  else:
    out_refs[0][...] = hn[:, :HH]
    out_refs[1][...] = hn[:, HH:]


def _tc_layer(h0, h1, agg2, cnt2, wl, wr, bb, g, bn, last):
  if last:
    out_specs = [
        pl.BlockSpec((BLK, H), lambda i: (i, 0)),
        pl.BlockSpec((1, H), lambda i: (0, 0)),
    ]
    out_shape = [
        jax.ShapeDtypeStruct((N, H), jnp.float32),
        jax.ShapeDtypeStruct((1, H), jnp.float32),
    ]
  else:
    out_specs = [
        pl.BlockSpec((BLK, HH), lambda i: (i, 0)),
        pl.BlockSpec((BLK, HH), lambda i: (i, 0)),
    ]
    out_shape = [
        jax.ShapeDtypeStruct((N, HH), jnp.float32),
        jax.ShapeDtypeStruct((N, HH), jnp.float32),
    ]
  return pl.pallas_call(
      functools.partial(_layer_body, last=last),
      grid=(GRID,),
      in_specs=[
          pl.BlockSpec((BLK, HH), lambda i: (i, 0)),
          pl.BlockSpec((BLK, HH), lambda i: (i, 0)),
          pl.BlockSpec((BLK, 128), lambda i: (i, 0)),
          pl.BlockSpec((1, BLK, 16), lambda i: (0, i, 0)),
          pl.BlockSpec((1, BLK, 16), lambda i: (1, i, 0)),
          pl.BlockSpec((H, H), lambda i: (0, 0)),
          pl.BlockSpec((H, H), lambda i: (0, 0)),
          pl.BlockSpec((1, H), lambda i: (0, 0)),
          pl.BlockSpec((1, H), lambda i: (0, 0)),
          pl.BlockSpec((1, H), lambda i: (0, 0)),
      ],
      out_specs=out_specs,
      out_shape=out_shape,
  )(h0, h1, agg2, cnt2, cnt2, wl, wr, bb, g, bn)


def _gmlp_body(gs_ref, w1_ref, b1_ref, w2_ref, b2_ref, iw1_ref, ib1_ref,
               iw2_ref, ib2_ref, g_ref, imp_ref):
  g0 = jnp.broadcast_to(gs_ref[...] * (1.0 / N), (8, H))
  a = jnp.maximum(_mm(g0, w1_ref[...]) + b1_ref[...], 0.0)
  g = _mm(a, w2_ref[...]) + b2_ref[...]
  ib = jnp.maximum(_mm(g, iw1_ref[...]) + ib1_ref[...], 0.0)
  imp = _mm(ib, iw2_ref[...]) + ib2_ref[...]
  g_ref[...] = g[:1]
  imp_ref[...] = imp[:1]


def _tc_gmlp(gsum, p):
  return pl.pallas_call(
      _gmlp_body,
      out_shape=[
          jax.ShapeDtypeStruct((1, H), jnp.float32),
          jax.ShapeDtypeStruct((1, 3), jnp.float32),
      ],
  )(gsum, p['gp_W1'], p['gp_b1'].reshape(1, -1), p['gp_W2'],
    p['gp_b2'].reshape(1, -1), p['im_W1'], p['im_b1'].reshape(1, -1),
    p['im_W2'], p['im_b2'].reshape(1, -1))


def _heads_body(h_ref, g_ref, sw1_ref, sb1_ref, sw2_ref, sb2_ref,
                ewh_ref, ewg_ref, eb1_ref, ew2_ref, eb2_ref,
                sev_ref, sp_ref):
  h = h_ref[...]
  s1 = jnp.maximum(_mm(h, sw1_ref[...]) + sb1_ref[...], 0.0)
  sev_ref[...] = jnp.maximum(_mm(s1, sw2_ref[...]) + sb2_ref[...], 0.0)
  gg = _mm(g_ref[...], ewg_ref[...]) + eb1_ref[...]
  e1 = jnp.maximum(_mm(h, ewh_ref[...]) + gg, 0.0)
  z = _mm(e1, ew2_ref[...]) + eb2_ref[...]
  sp_ref[...] = 1.0 / (1.0 + jnp.exp(-z))


def _tc_heads(hf, g, p):
  return pl.pallas_call(
      _heads_body,
      grid=(GRID,),
      in_specs=[
          pl.BlockSpec((BLK, H), lambda i: (i, 0)),
          pl.BlockSpec((1, H), lambda i: (0, 0)),
          pl.BlockSpec((H, HH), lambda i: (0, 0)),
          pl.BlockSpec((1, HH), lambda i: (0, 0)),
          pl.BlockSpec((HH, 1), lambda i: (0, 0)),
          pl.BlockSpec((1, 1), lambda i: (0, 0)),
          pl.BlockSpec((H, HH), lambda i: (0, 0)),
          pl.BlockSpec((H, HH), lambda i: (0, 0)),
          pl.BlockSpec((1, HH), lambda i: (0, 0)),
          pl.BlockSpec((HH, 1), lambda i: (0, 0)),
          pl.BlockSpec((1, 1), lambda i: (0, 0)),
      ],
      out_specs=[
          pl.BlockSpec((BLK, 1), lambda i: (i, 0)),
          pl.BlockSpec((BLK, 1), lambda i: (i, 0)),
      ],
      out_shape=[
          jax.ShapeDtypeStruct((N, 1), jnp.float32),
          jax.ShapeDtypeStruct((N, 1), jnp.float32),
      ],
  )(hf, g, p['sv_W1'], p['sv_b1'].reshape(1, -1), p['sv_W2'],
    p['sv_b2'].reshape(1, -1), p['se_W1'][:H], p['se_W1'][H:],
    p['se_b1'].reshape(1, -1), p['se_W2'], p['se_b2'].reshape(1, -1))


def kernel(x, params, edge_index):
  p = params
  src = edge_index[0]
  dst = edge_index[1]

  pad_a = E_PAD - E
  src_p = jnp.concatenate(
      [src, (jnp.arange(pad_a, dtype=jnp.int32) * 127) % N])
  dst_p = jnp.concatenate(
      [dst, N + jnp.arange(pad_a, dtype=jnp.int32) % PADR])
  pad_c = E_CNT - E
  dst_c = jnp.concatenate(
      [dst, N + jnp.arange(pad_c, dtype=jnp.int32) % PADR])

  zeros32 = jnp.zeros((ACC_ROWS, HH), jnp.float32)
  zeros16 = jnp.zeros((ACC_ROWS, 16), jnp.float32)
  ones16 = jnp.ones((CNT_C, 16), jnp.float32)

  cnt2 = _sc_cnt(dst_c, zeros16, ones16)
  h0, h1 = _tc_proj(x, p['in_W'], p['in_b'].reshape(1, -1))

  hf = gsum = None
  for i in range(3):
    agg2 = _sc_agg(h0, h1, src_p, dst_p, zeros32)
    bb = (p['l%d_bl' % i] + p['l%d_br' % i]).reshape(1, -1)
    outs = _tc_layer(h0, h1, agg2, cnt2, p['l%d_Wl' % i], p['l%d_Wr' % i],
                     bb, p['l%d_g' % i].reshape(1, -1),
                     p['l%d_bn' % i].reshape(1, -1), last=(i == 2))
    if i == 2:
      hf, gsum = outs
    else:
      h0, h1 = outs

  g, imp = _tc_gmlp(gsum, p)
  sev, sp = _tc_heads(hf, g, p)
  return (sev, imp, sp, hf, g)
